# speedup vs baseline: 1.8911x; 1.8911x over previous
import functools

import jax
import jax.numpy as jnp
from jax import lax
from jax.experimental import pallas as pl
from jax.experimental.pallas import tpu as pltpu
from jax.experimental.pallas import tpu_sc as plsc

POOL = 1000
KDIM = 3840
PDIM = 13824
BATCH = 1024
TOPK = 2
NTOK = 4

BM = 128
GRID = BATCH // BM

NC, NS = 2, 16
NW = NC * NS
NIDX = BATCH * TOPK
BPW = NIDX // NW
CH = 4
NCHUNK = BPW // CH


def _sim_body(x_ref, pk_ref, sim_ref, idx_ref, acc_ref):
    i = pl.program_id(0)
    x = x_ref[...]
    xm = (x[:, 0, :] + x[:, 1, :] + x[:, 2, :] + x[:, 3, :]) * (1.0 / NTOK)
    xsq = jnp.sum(xm * xm, axis=1, keepdims=True)
    xn = xm * lax.rsqrt(jnp.maximum(xsq, 1e-12))
    pk = pk_ref[...]
    psq = jnp.sum(pk * pk, axis=1, keepdims=True)
    pn = pk * lax.rsqrt(jnp.maximum(psq, 1e-12))
    sim = lax.dot_general(xn, pn, (((1,), (1,)), ((), ())),
                          preferred_element_type=jnp.float32)
    sim_ref[...] = sim

    cols = lax.broadcasted_iota(jnp.int32, (BM, POOL), 1)
    m1 = jnp.max(sim, axis=1)
    a1 = jnp.min(jnp.where(sim == m1[:, None], cols, POOL), axis=1)
    simm = jnp.where(cols == a1[:, None], -jnp.inf, sim)
    m2 = jnp.max(simm, axis=1)
    a2 = jnp.min(jnp.where(simm == m2[:, None], cols, POOL), axis=1)
    idx_ref[...] = jnp.concatenate([a1[:, None], a2[:, None]], axis=1)

    @pl.when(i == 0)
    def _():
        acc_ref[0, 0] = 0.0

    acc_ref[0, 0] += jnp.sum(m1) + jnp.sum(m2)


def _sim_call(x_embed, prompt_key):
    return pl.pallas_call(
        _sim_body,
        grid=(GRID,),
        in_specs=[
            pl.BlockSpec((BM, NTOK, KDIM), lambda i: (i, 0, 0)),
            pl.BlockSpec((POOL, KDIM), lambda i: (0, 0)),
        ],
        out_specs=[
            pl.BlockSpec((BM, POOL), lambda i: (i, 0)),
            pl.BlockSpec((BM, TOPK), lambda i: (i, 0)),
            pl.BlockSpec(block_shape=(1, 1), index_map=lambda i: (0, 0),
                         memory_space=pltpu.SMEM),
        ],
        out_shape=[
            jax.ShapeDtypeStruct((BATCH, POOL), jnp.float32),
            jax.ShapeDtypeStruct((BATCH, TOPK), jnp.int32),
            jax.ShapeDtypeStruct((1, 1), jnp.float32),
        ],
        compiler_params=pltpu.CompilerParams(
            dimension_semantics=("arbitrary",)),
    )(x_embed, prompt_key)


def _gather_body(table, idxm, out, idx_v, buf0, buf1,
                 gsem0, gsem1, ssem0, ssem1):
    c = lax.axis_index("c")
    s = lax.axis_index("s")
    wid = s * NC + c
    pltpu.sync_copy(idxm.at[pl.ds(wid * NCHUNK, NCHUNK)], idx_v)

    bufs = (buf0, buf1)
    gsems = (gsem0, gsem1)
    ssems = (ssem0, ssem1)

    def start_gather(g):
        b = g & 1
        return pltpu.async_copy(table.at[idx_v.at[g]], bufs[b], gsems[b])

    def start_scatter(g):
        b = g & 1
        base = wid * BPW + g * CH
        return pltpu.async_copy(bufs[b], out.at[pl.ds(base, CH)], ssems[b])

    gcopies = [start_gather(0), start_gather(1)]
    scopies = [None, None]
    for g in range(NCHUNK):
        b = g & 1
        gcopies[b].wait()
        scopies[b] = start_scatter(g)
        if g + 2 < NCHUNK:
            scopies[b].wait()
            gcopies[b] = start_gather(g + 2)
    scopies[0].wait()
    scopies[1].wait()


def _gather_call(table, idxm):
    mesh = plsc.VectorSubcoreMesh(core_axis_name="c", subcore_axis_name="s",
                                  num_cores=NC, num_subcores=NS)
    run = pl.kernel(
        _gather_body,
        out_type=jax.ShapeDtypeStruct((NIDX, PDIM), jnp.float32),
        mesh=mesh,
        scratch_types=[
            pltpu.VMEM((NCHUNK, CH), jnp.int32),
            pltpu.VMEM((CH, PDIM), jnp.float32),
            pltpu.VMEM((CH, PDIM), jnp.float32),
            pltpu.SemaphoreType.DMA,
            pltpu.SemaphoreType.DMA,
            pltpu.SemaphoreType.DMA,
            pltpu.SemaphoreType.DMA,
        ],
    )
    return run(table, idxm)


def kernel(x_embed, prompt, prompt_key):
    sim, idx, acc = _sim_call(x_embed, prompt_key)
    idxm = idx.reshape(NW * NCHUNK, CH)
    table = prompt.reshape(POOL, PDIM)
    rows = _gather_call(table, idxm)
    batched_prompt = rows.reshape(BATCH, TOPK, PDIM)
    reduce_sim = acc[0, 0] / BATCH
    return batched_prompt, sim, idx, reduce_sim
